# baseline (speedup 1.0000x reference)
import functools

import jax
import jax.numpy as jnp
from jax import lax
from jax.experimental import pallas as pl
from jax.experimental.pallas import tpu as pltpu
from jax.experimental.pallas import tpu_sc as plsc

_LAMDA = 0.01
_B = 16384
_D = 16

_NC, _NS = 2, 16
_NW = _NC * _NS
_BPW = _B // _NW
_CHUNK = 128
_NCHUNK = _BPW // _CHUNK
_NBLK = _BPW // _D


def _sc_part(eu_t, ei_t, idx2d):
    mesh = plsc.VectorSubcoreMesh(core_axis_name="c", subcore_axis_name="s")
    out_type = [jax.ShapeDtypeStruct((_B,), jnp.float32) for _ in range(7)]
    scratch = (
        [pltpu.VMEM((_NCHUNK, _CHUNK), jnp.int32) for _ in range(6)]
        + [pltpu.VMEM((_D, _BPW), jnp.float32) for _ in range(6)]
        + [pltpu.VMEM((_BPW,), jnp.float32) for _ in range(7)]
        + [pltpu.SemaphoreType.DMA for _ in range(6)]
    )

    @functools.partial(
        pl.kernel, mesh=mesh, out_type=out_type, scratch_types=scratch,
        compiler_params=pltpu.CompilerParams(
            use_tc_tiling_on_sc=False, needs_layout_passes=False
        ),
    )
    def body(eu, ei, idx_hbm,
             xo, q0, q1, q2, q3, q4, q5,
             iv0, iv1, iv2, iv3, iv4, iv5,
             cv0, cv1, cv2, cv3, cv4, cv5,
             xv, w0, w1, w2, w3, w4, w5,
             s0, s1, s2, s3, s4, s5):
        ivs = (iv0, iv1, iv2, iv3, iv4, iv5)
        cvs = (cv0, cv1, cv2, cv3, cv4, cv5)
        wvs = (w0, w1, w2, w3, w4, w5)
        qouts = (q0, q1, q2, q3, q4, q5)
        sems = (s0, s1, s2, s3, s4, s5)
        tables = (eu, ei, eu, ei, eu, ei)

        wid = lax.axis_index("s") * _NC + lax.axis_index("c")
        crow = wid * _NCHUNK

        for t in range(6):
            pltpu.sync_copy(idx_hbm.at[t, pl.ds(crow, _NCHUNK)], ivs[t])

        handles = []
        for t in range(6):
            for d in range(_D):
                for j in range(_NCHUNK):
                    handles.append(
                        pltpu.async_copy(
                            tables[t].at[d].at[ivs[t].at[j]],
                            cvs[t].at[d, pl.ds(j * _CHUNK, _CHUNK)],
                            sems[t],
                        )
                    )
        for h in handles:
            h.wait()

        def block(m, _):
            s = pl.ds(m * _D, _D)
            zero = jnp.zeros((16,), jnp.float32)
            x = zero
            qs = [zero] * 6
            for d in range(_D):
                c = [cvs[t][d, s] for t in range(6)]
                g = c[0] + c[1]
                gp = c[2] + c[3]
                gn = c[4] + c[5]
                x = x + g * (gp - gn)
                for t in range(6):
                    qs[t] = qs[t] + c[t] * c[t]
            xv[s] = x
            for t in range(6):
                wvs[t][s] = qs[t]
            return _

        lax.fori_loop(0, _NBLK, block, None)

        base = wid * _BPW
        pltpu.sync_copy(xv, xo.at[pl.ds(base, _BPW)])
        for t in range(6):
            pltpu.sync_copy(wvs[t], qouts[t].at[pl.ds(base, _BPW)])

    return body(eu_t, ei_t, idx2d)


def _tc_reduce(x, q0, q1, q2, q3, q4, q5):
    def body(x_r, q0_r, q1_r, q2_r, q3_r, q4_r, q5_r, loss_r, reg_r):
        x = x_r[...]
        loss_r[0, 0] = jnp.sum(jnp.log1p(jnp.exp(-x)))
        reg = 0.0
        for q in (q0_r, q1_r, q2_r, q3_r, q4_r, q5_r):
            reg = reg + jnp.sum(jnp.sqrt(q[...]))
        reg_r[0, 0] = reg * _LAMDA

    loss, reg = pl.pallas_call(
        body,
        out_shape=[jax.ShapeDtypeStruct((1, 1), jnp.float32)] * 2,
        in_specs=[pl.BlockSpec(memory_space=pltpu.VMEM)] * 7,
        out_specs=[pl.BlockSpec(memory_space=pltpu.SMEM)] * 2,
    )(x, q0, q1, q2, q3, q4, q5)
    return loss[0, 0], reg[0, 0]


def kernel(embed_user, embed_item, user, item, pos_user, pos_item, neg_user, neg_item):
    idx2d = jnp.stack(
        [user, item, pos_user, pos_item, neg_user, neg_item]
    ).reshape(6, _B // _CHUNK, _CHUNK)
    x, q0, q1, q2, q3, q4, q5 = _sc_part(embed_user.T, embed_item.T, idx2d)
    sq = _B // _CHUNK
    return _tc_reduce(
        x.reshape(sq, _CHUNK),
        q0.reshape(sq, _CHUNK),
        q1.reshape(sq, _CHUNK),
        q2.reshape(sq, _CHUNK),
        q3.reshape(sq, _CHUNK),
        q4.reshape(sq, _CHUNK),
        q5.reshape(sq, _CHUNK),
    )


# speedup vs baseline: 3.1784x; 3.1784x over previous
import functools

import jax
import jax.numpy as jnp
from jax import lax
from jax.experimental import pallas as pl
from jax.experimental.pallas import tpu as pltpu
from jax.experimental.pallas import tpu_sc as plsc

_LAMDA = 0.01
_B = 16384
_D = 16

_NC, _NS = 2, 16
_NW = _NC * _NS
_BPW = _B // _NW
_CHUNK = 128
_NCHUNK = _BPW // _CHUNK
_NBLK = _BPW // _D


def _sc_part(embed_user, embed_item, idx2d):
    mesh = plsc.VectorSubcoreMesh(core_axis_name="c", subcore_axis_name="s")
    out_type = [jax.ShapeDtypeStruct((_B,), jnp.float32) for _ in range(7)]
    scratch = (
        [pltpu.VMEM((_NCHUNK, _CHUNK), jnp.int32) for _ in range(6)]
        + [pltpu.VMEM((_BPW, _D), jnp.float32) for _ in range(6)]
        + [pltpu.VMEM((_BPW,), jnp.float32) for _ in range(7)]
        + [pltpu.SemaphoreType.DMA for _ in range(6)]
    )

    @functools.partial(
        pl.kernel, mesh=mesh, out_type=out_type, scratch_types=scratch,
        compiler_params=pltpu.CompilerParams(
            use_tc_tiling_on_sc=False, needs_layout_passes=False
        ),
    )
    def body(eu, ei, idx_hbm,
             xo, q0, q1, q2, q3, q4, q5,
             iv0, iv1, iv2, iv3, iv4, iv5,
             rv0, rv1, rv2, rv3, rv4, rv5,
             xv, w0, w1, w2, w3, w4, w5,
             s0, s1, s2, s3, s4, s5):
        ivs = (iv0, iv1, iv2, iv3, iv4, iv5)
        rvs = (rv0, rv1, rv2, rv3, rv4, rv5)
        wvs = (w0, w1, w2, w3, w4, w5)
        qouts = (q0, q1, q2, q3, q4, q5)
        sems = (s0, s1, s2, s3, s4, s5)
        tables = (eu, ei, eu, ei, eu, ei)

        wid = lax.axis_index("s") * _NC + lax.axis_index("c")
        crow = wid * _NCHUNK

        for t in range(6):
            pltpu.sync_copy(idx_hbm.at[t, pl.ds(crow, _NCHUNK)], ivs[t])

        handles = []
        for t in range(6):
            for j in range(_NCHUNK):
                handles.append(
                    pltpu.async_copy(
                        tables[t].at[ivs[t].at[j]],
                        rvs[t].at[pl.ds(j * _CHUNK, _CHUNK)],
                        sems[t],
                    )
                )
        for h in handles:
            h.wait()

        lane = lax.iota(jnp.int32, 16)

        def block(m, _):
            row = m * _D + lane
            cols = []
            for t in range(6):
                ct = []
                for d in range(_D):
                    col = jnp.full((16,), d, dtype=jnp.int32)
                    ct.append(plsc.load_gather(rvs[t], [row, col]))
                cols.append(ct)
            zero = jnp.zeros((16,), jnp.float32)
            x = zero
            qs = [zero] * 6
            for d in range(_D):
                g = cols[0][d] + cols[1][d]
                gp = cols[2][d] + cols[3][d]
                gn = cols[4][d] + cols[5][d]
                x = x + g * (gp - gn)
                for t in range(6):
                    qs[t] = qs[t] + cols[t][d] * cols[t][d]
            xv[pl.ds(m * _D, _D)] = x
            for t in range(6):
                wvs[t][pl.ds(m * _D, _D)] = qs[t]
            return _

        lax.fori_loop(0, _NBLK, block, None)

        base = wid * _BPW
        pltpu.sync_copy(xv, xo.at[pl.ds(base, _BPW)])
        for t in range(6):
            pltpu.sync_copy(wvs[t], qouts[t].at[pl.ds(base, _BPW)])

    return body(embed_user, embed_item, idx2d)


def _tc_reduce(x, q0, q1, q2, q3, q4, q5):
    def body(x_r, q0_r, q1_r, q2_r, q3_r, q4_r, q5_r, loss_r, reg_r):
        x = x_r[...]
        loss_r[0, 0] = jnp.sum(jnp.log1p(jnp.exp(-x)))
        reg = 0.0
        for q in (q0_r, q1_r, q2_r, q3_r, q4_r, q5_r):
            reg = reg + jnp.sum(jnp.sqrt(q[...]))
        reg_r[0, 0] = reg * _LAMDA

    loss, reg = pl.pallas_call(
        body,
        out_shape=[jax.ShapeDtypeStruct((1, 1), jnp.float32)] * 2,
        in_specs=[pl.BlockSpec(memory_space=pltpu.VMEM)] * 7,
        out_specs=[pl.BlockSpec(memory_space=pltpu.SMEM)] * 2,
    )(x, q0, q1, q2, q3, q4, q5)
    return loss[0, 0], reg[0, 0]


def kernel(embed_user, embed_item, user, item, pos_user, pos_item, neg_user, neg_item):
    idx2d = jnp.stack(
        [user, item, pos_user, pos_item, neg_user, neg_item]
    ).reshape(6, _B // _CHUNK, _CHUNK)
    x, q0, q1, q2, q3, q4, q5 = _sc_part(embed_user, embed_item, idx2d)
    sq = _B // _CHUNK
    return _tc_reduce(
        x.reshape(sq, _CHUNK),
        q0.reshape(sq, _CHUNK),
        q1.reshape(sq, _CHUNK),
        q2.reshape(sq, _CHUNK),
        q3.reshape(sq, _CHUNK),
        q4.reshape(sq, _CHUNK),
        q5.reshape(sq, _CHUNK),
    )
